# Initial kernel scaffold; baseline (speedup 1.0000x reference)
import jax
import jax.numpy as jnp
from jax.experimental import pallas as pl


def kernel(X, edge_index, edge_weight, H, Wz, bz, Wr, br, Wh, bh, Lz_w, Lz_b, Lr_w, Lr_b, Lh_w, Lh_b):
    raise NotImplementedError("write your pallas kernel here")


# speedup vs baseline: 2.9291x; 2.9291x over previous
import functools

import jax
import jax.numpy as jnp
from jax import lax
from jax.experimental import pallas as pl
from jax.experimental.pallas import tpu as pltpu
from jax.experimental.pallas import tpu_sc as plsc

_N = 10000
_E = 320000
_D = 128
_NP = 10240
_EP = 327680
_NCORE = 2
_NSUB = 16
_K = 128
_CPT = (_EP // (_NCORE * _NSUB)) // _K
_PER_SUB = _NP // _NSUB

def _deg_body(edata, ew, out, edv, ewv, deg_o, deg_i, slab, acc, tbuf):
    c = lax.axis_index("c")
    s = lax.axis_index("s")
    wid = s * _NCORE + c
    pltpu.sync_copy(edata.at[pl.ds(wid * _CPT, _CPT)], edv)
    pltpu.sync_copy(ew.at[pl.ds(wid * _CPT * _K, _CPT * _K)], ewv)

    zero = jnp.zeros((16,), jnp.float32)

    def zb(i, _):
        deg_o[pl.ds(i * 16, 16)] = zero
        deg_i[pl.ds(i * 16, 16)] = zero
        return 0
    lax.fori_loop(0, _NP // 16, zb, 0)

    def eb(m, _):
        for i in range(_K // 16):
            sl = pl.ds(i * 16, 16)
            si = edv[m, 0, sl]
            di = edv[m, 1, sl]
            w = ewv[pl.ds(m * _K + i * 16, 16)]
            plsc.addupdate_scatter(deg_o, [si], w)
            plsc.addupdate_scatter(deg_i, [di], w)
        return 0
    lax.fori_loop(0, _CPT, eb, 0)

    pltpu.sync_copy(deg_o, slab.at[s, 0])
    pltpu.sync_copy(deg_i, slab.at[s, 1])
    plsc.subcore_barrier()

    nb = _PER_SUB // 16

    def za(i, _):
        acc[0, pl.ds(i * 16, 16)] = zero
        acc[1, pl.ds(i * 16, 16)] = zero
        return 0
    lax.fori_loop(0, nb, za, 0)

    def rb(t, _):
        pltpu.sync_copy(slab.at[t, :, pl.ds(s * _PER_SUB, _PER_SUB)], tbuf)

        def ab(b, _):
            sl = pl.ds(b * 16, 16)
            acc[0, sl] = acc[0, sl] + tbuf[0, sl]
            acc[1, sl] = acc[1, sl] + tbuf[1, sl]
            return 0
        lax.fori_loop(0, nb, ab, 0)
        return 0
    lax.fori_loop(0, _NSUB, rb, 0)

    pltpu.sync_copy(acc, out.at[c, :, pl.ds(s * _PER_SUB, _PER_SUB)])


def _agg_body(xs, edata, ew, out, edv, ewv, rows, agg_sh, sem):
    c = lax.axis_index("c")
    s = lax.axis_index("s")
    wid = s * _NCORE + c
    pltpu.sync_copy(edata.at[pl.ds(wid * _CPT, _CPT)], edv)
    pltpu.sync_copy(ew.at[pl.ds(wid * _CPT * _K, _CPT * _K)], ewv)

    zero = jnp.zeros((16,), jnp.float32)

    def zb(j, _):
        for v in range(_D // 16):
            rows[j, pl.ds(v * 16, 16)] = zero
        return 0
    lax.fori_loop(0, _K, zb, 0)
    for jj in range(_PER_SUB // _K):
        pltpu.sync_copy(rows, agg_sh.at[pl.ds(s * _PER_SUB + jj * _K, _K)])
    plsc.subcore_barrier()

    def mb(m, _):
        pltpu.async_copy(xs.at[edv.at[m, 0]], rows, sem).wait()

        def jb(j, _):
            w = plsc.load_gather(ewv, [jnp.full((16,), m * _K + j, jnp.int32)])
            for v in range(_D // 16):
                sl = pl.ds(v * 16, 16)
                rows[j, sl] = rows[j, sl] * w
            return 0
        lax.fori_loop(0, _K, jb, 0)

        pltpu.sync_copy(rows, agg_sh.at[edv.at[m, 1]], add=True)
        return 0
    lax.fori_loop(0, _CPT, mb, 0)

    plsc.subcore_barrier()
    for jj in range(_PER_SUB // _K):
        sl = pl.ds(s * _PER_SUB + jj * _K, _K)
        pltpu.sync_copy(agg_sh.at[sl], out.at[c, sl])


def _prep_body(dref, xref, xsref, ndref):
    d = dref[...]
    dego = d[0] + d[2]
    degi = d[1] + d[3]
    nsrc = jnp.where(dego > 0, lax.rsqrt(jnp.where(dego > 0, dego, 1.0)), 0.0)
    ndst = jnp.where(degi > 0, lax.rsqrt(jnp.where(degi > 0, degi, 1.0)), 0.0)
    xsref[...] = xref[...] * nsrc
    ndref[...] = ndst


_prep_call = pl.pallas_call(
    _prep_body,
    grid=(_NP // 128,),
    in_specs=[
        pl.BlockSpec((4, 128, 1), lambda i: (0, i, 0)),
        pl.BlockSpec((128, _D), lambda i: (i, 0)),
    ],
    out_specs=[
        pl.BlockSpec((128, _D), lambda i: (i, 0)),
        pl.BlockSpec((128, 1), lambda i: (i, 0)),
    ],
    out_shape=[
        jax.ShapeDtypeStruct((_NP, _D), jnp.float32),
        jax.ShapeDtypeStruct((_NP, 1), jnp.float32),
    ],
)


def _fin_body(aref, ndref, href, wz, bz, wr, br, wh, bh,
              lz, lzb, lr, lrb, lh, lhb, oref):
    f32 = jnp.float32
    agg = aref[0] + aref[1]
    rst = agg * ndref[...]
    h = href[...]
    hz = jnp.dot(rst, wz[...], preferred_element_type=f32) + bz[...]
    hr = jnp.dot(rst, wr[...], preferred_element_type=f32) + br[...]
    hh = jnp.dot(rst, wh[...], preferred_element_type=f32) + bh[...]
    lzm = lz[...]
    z = jax.nn.sigmoid(jnp.dot(hz, lzm[:_D], preferred_element_type=f32)
                       + jnp.dot(h, lzm[_D:], preferred_element_type=f32)
                       + lzb[...])
    lrm = lr[...]
    r = jax.nn.sigmoid(jnp.dot(hr, lrm[:_D], preferred_element_type=f32)
                       + jnp.dot(h, lrm[_D:], preferred_element_type=f32)
                       + lrb[...])
    lhm = lh[...]
    ht = jnp.tanh(jnp.dot(hh, lhm[:_D], preferred_element_type=f32)
                  + jnp.dot(h * r, lhm[_D:], preferred_element_type=f32)
                  + lhb[...])
    oref[...] = z * h + (1.0 - z) * ht


def _full(shape):
    return pl.BlockSpec(shape, lambda i: tuple(0 for _ in shape))


_fin_call = pl.pallas_call(
    _fin_body,
    grid=(_NP // 128,),
    in_specs=[
        pl.BlockSpec((2, 128, _D), lambda i: (0, i, 0)),
        pl.BlockSpec((128, 1), lambda i: (i, 0)),
        pl.BlockSpec((128, _D), lambda i: (i, 0)),
        _full((_D, _D)), _full((1, _D)),
        _full((_D, _D)), _full((1, _D)),
        _full((_D, _D)), _full((1, _D)),
        _full((2 * _D, _D)), _full((1, _D)),
        _full((2 * _D, _D)), _full((1, _D)),
        _full((2 * _D, _D)), _full((1, _D)),
    ],
    out_specs=pl.BlockSpec((128, _D), lambda i: (i, 0)),
    out_shape=jax.ShapeDtypeStruct((_NP, _D), jnp.float32),
)


@functools.cache
def _sc_calls():
    mesh = plsc.VectorSubcoreMesh(
        core_axis_name="c", subcore_axis_name="s", num_cores=_NCORE,
        num_subcores=_NSUB)
    cparams = pltpu.CompilerParams(needs_layout_passes=False)
    deg_call = pl.kernel(
        _deg_body,
        out_type=jax.ShapeDtypeStruct((_NCORE, 2, _NP), jnp.float32),
        mesh=mesh,
        compiler_params=cparams,
        scratch_types=[
            pltpu.VMEM((_CPT, 2, _K), jnp.int32),
            pltpu.VMEM((_CPT * _K,), jnp.float32),
            pltpu.VMEM((_NP,), jnp.float32),
            pltpu.VMEM((_NP,), jnp.float32),
            pltpu.VMEM_SHARED((_NSUB, 2, _NP), jnp.float32),
            pltpu.VMEM((2, _PER_SUB), jnp.float32),
            pltpu.VMEM((2, _PER_SUB), jnp.float32),
        ],
    )
    agg_call = pl.kernel(
        _agg_body,
        out_type=jax.ShapeDtypeStruct((_NCORE, _NP, _D), jnp.float32),
        mesh=mesh,
        compiler_params=cparams,
        scratch_types=[
            pltpu.VMEM((_CPT, 2, _K), jnp.int32),
            pltpu.VMEM((_CPT * _K,), jnp.float32),
            pltpu.VMEM((_K, _D), jnp.float32),
            pltpu.VMEM_SHARED((_NP, _D), jnp.float32),
            pltpu.SemaphoreType.DMA,
        ],
    )
    return deg_call, agg_call


def kernel(X, edge_index, edge_weight, H, Wz, bz, Wr, br, Wh, bh,
           Lz_w, Lz_b, Lr_w, Lr_b, Lh_w, Lh_b):
    _deg_call, _agg_call = _sc_calls()
    pe = _EP - _E
    srcp = jnp.concatenate([edge_index[0], jnp.zeros((pe,), jnp.int32)])
    dstp = jnp.concatenate([edge_index[1], jnp.zeros((pe,), jnp.int32)])
    ewp = jnp.concatenate([edge_weight, jnp.zeros((pe,), jnp.float32)])
    edata = jnp.stack(
        [srcp.reshape(-1, _K), dstp.reshape(-1, _K)], axis=1)
    Xp = jnp.pad(X, ((0, _NP - _N), (0, 0)))
    Hp = jnp.pad(H, ((0, _NP - _N), (0, 0)))

    degs = _deg_call(edata, ewp)
    xs, ndst = _prep_call(degs.reshape(4, _NP, 1), Xp)
    agg = _agg_call(xs, edata, ewp)

    rb = lambda b: b.reshape(1, _D)
    out = _fin_call(agg, ndst, Hp, Wz, rb(bz), Wr, rb(br), Wh, rb(bh),
                    Lz_w, rb(Lz_b), Lr_w, rb(Lr_b), Lh_w, rb(Lh_b))
    return out[:_N]


# speedup vs baseline: 3.5170x; 1.2007x over previous
import functools

import jax
import jax.numpy as jnp
from jax import lax
from jax.experimental import pallas as pl
from jax.experimental.pallas import tpu as pltpu
from jax.experimental.pallas import tpu_sc as plsc

_N = 10000
_E = 320000
_D = 128
_NP = 10240
_EP = 327680
_NCORE = 2
_NSUB = 16
_K = 128
_CPT = (_EP // (_NCORE * _NSUB)) // _K
_PER_SUB = _NP // _NSUB

def _deg_body(edata, ew, out, edv, ewv, deg_o, deg_i, slab, acc, tbuf):
    c = lax.axis_index("c")
    s = lax.axis_index("s")
    wid = s * _NCORE + c
    pltpu.sync_copy(edata.at[pl.ds(wid * _CPT, _CPT)], edv)
    pltpu.sync_copy(ew.at[pl.ds(wid * _CPT * _K, _CPT * _K)], ewv)

    zero = jnp.zeros((16,), jnp.float32)

    def zb(i, _):
        deg_o[pl.ds(i * 16, 16)] = zero
        deg_i[pl.ds(i * 16, 16)] = zero
        return 0
    lax.fori_loop(0, _NP // 16, zb, 0)

    def eb(m, _):
        for i in range(_K // 16):
            sl = pl.ds(i * 16, 16)
            si = edv[m, 0, sl]
            di = edv[m, 1, sl]
            w = ewv[pl.ds(m * _K + i * 16, 16)]
            plsc.addupdate_scatter(deg_o, [si], w)
            plsc.addupdate_scatter(deg_i, [di], w)
        return 0
    lax.fori_loop(0, _CPT, eb, 0)

    pltpu.sync_copy(deg_o, slab.at[s, 0])
    pltpu.sync_copy(deg_i, slab.at[s, 1])
    plsc.subcore_barrier()

    nb = _PER_SUB // 16

    def za(i, _):
        acc[0, pl.ds(i * 16, 16)] = zero
        acc[1, pl.ds(i * 16, 16)] = zero
        return 0
    lax.fori_loop(0, nb, za, 0)

    def rb(t, _):
        pltpu.sync_copy(slab.at[t, :, pl.ds(s * _PER_SUB, _PER_SUB)], tbuf)

        def ab(b, _):
            sl = pl.ds(b * 16, 16)
            acc[0, sl] = acc[0, sl] + tbuf[0, sl]
            acc[1, sl] = acc[1, sl] + tbuf[1, sl]
            return 0
        lax.fori_loop(0, nb, ab, 0)
        return 0
    lax.fori_loop(0, _NSUB, rb, 0)

    pltpu.sync_copy(acc, out.at[c, :, pl.ds(s * _PER_SUB, _PER_SUB)])


_NB = 2
_DR = 4
_U = 4


def _agg_body(xs, edata, ew, out, desc, ewr, rows, agg_sh,
              g0, g1, s0, s1, d0, d1, d2, d3, w0, w1, w2, w3):
    c = lax.axis_index("c")
    s = lax.axis_index("s")
    wid = s * _NCORE + c
    sem_g = (g0, g1)
    sem_s = (s0, s1)
    sem_d = (d0, d1, d2, d3)
    sem_w = (w0, w1, w2, w3)
    cbase = wid * _CPT

    zero = jnp.zeros((16,), jnp.float32)

    def zb(j, _):
        for v in range(_D // 16):
            rows[0, j, pl.ds(v * 16, 16)] = zero
        return 0
    lax.fori_loop(0, _K, zb, 0)
    for jj in range(_PER_SUB // _K):
        pltpu.sync_copy(rows.at[0],
                        agg_sh.at[pl.ds(s * _PER_SUB + jj * _K, _K)])
    plsc.subcore_barrier()

    def desc_cp(m, d):
        return pltpu.make_async_copy(edata.at[cbase + m], desc.at[d],
                                     sem_d[d])

    def ew_cp(m, d):
        return pltpu.make_async_copy(ew.at[pl.ds((cbase + m) * _K, _K)],
                                     ewr.at[d], sem_w[d])

    def gather(m, b, d):
        return pltpu.make_async_copy(xs.at[desc.at[d, 0]], rows.at[b],
                                     sem_g[b])

    def scatter(m, b, d):
        return pltpu.make_async_copy(rows.at[b], agg_sh.at[desc.at[d, 1]],
                                     sem_s[b])

    for j in range(_DR - 1):
        desc_cp(j, j).start()
        ew_cp(j, j).start()
    desc_cp(0, 0).wait()
    gather(0, 0, 0).start()

    def pb(p, _):
        for q in range(_U):
            m = p * _U + q
            b = q % _NB
            d = q
            gather(m, b, d).wait()

            def adv():
                scatter(m - 1, 1 - b, (q - 1) % _DR).wait()

            if q > 0:
                adv()
            else:
                pl.when(p >= 1)(adv)

            def nxt():
                desc_cp(m + 1, (q + 1) % _DR).wait()
                gather(m + 1, 1 - b, (q + 1) % _DR).start()

            if q < _U - 1:
                nxt()
            else:
                pl.when(p < _CPT // _U - 1)(nxt)

            ew_cp(m, d).wait()

            def jb(j, _):
                w = plsc.load_gather(ewr.at[d],
                                     [jnp.full((16,), j, jnp.int32)])
                for v in range(_D // 16):
                    sl = pl.ds(v * 16, 16)
                    rows[b, j, sl] = rows[b, j, sl] * w
                return 0
            lax.fori_loop(0, _K, jb, 0)

            scatter(m, b, d).start(add=True)

            def refill():
                desc_cp(m + _DR - 1, (q + _DR - 1) % _DR).start()
                ew_cp(m + _DR - 1, (q + _DR - 1) % _DR).start()

            if q == 0:
                refill()
            else:
                pl.when(p < _CPT // _U - 1)(refill)
        return 0
    lax.fori_loop(0, _CPT // _U, pb, 0)
    scatter(_CPT - 1, (_CPT - 1) % _NB, (_CPT - 1) % _DR).wait()

    plsc.subcore_barrier()
    for jj in range(_PER_SUB // _K):
        sl = pl.ds(s * _PER_SUB + jj * _K, _K)
        pltpu.sync_copy(agg_sh.at[sl], out.at[c, sl])


def _prep_body(dref, xref, xsref, ndref):
    d = dref[...]
    dego = d[0] + d[2]
    degi = d[1] + d[3]
    nsrc = jnp.where(dego > 0, lax.rsqrt(jnp.where(dego > 0, dego, 1.0)), 0.0)
    ndst = jnp.where(degi > 0, lax.rsqrt(jnp.where(degi > 0, degi, 1.0)), 0.0)
    xsref[...] = xref[...] * nsrc
    ndref[...] = ndst


_prep_call = pl.pallas_call(
    _prep_body,
    grid=(_NP // 128,),
    in_specs=[
        pl.BlockSpec((4, 128, 1), lambda i: (0, i, 0)),
        pl.BlockSpec((128, _D), lambda i: (i, 0)),
    ],
    out_specs=[
        pl.BlockSpec((128, _D), lambda i: (i, 0)),
        pl.BlockSpec((128, 1), lambda i: (i, 0)),
    ],
    out_shape=[
        jax.ShapeDtypeStruct((_NP, _D), jnp.float32),
        jax.ShapeDtypeStruct((_NP, 1), jnp.float32),
    ],
)


def _fin_body(aref, ndref, href, wz, bz, wr, br, wh, bh,
              lz, lzb, lr, lrb, lh, lhb, oref):
    f32 = jnp.float32
    agg = aref[0] + aref[1]
    rst = agg * ndref[...]
    h = href[...]
    hz = jnp.dot(rst, wz[...], preferred_element_type=f32) + bz[...]
    hr = jnp.dot(rst, wr[...], preferred_element_type=f32) + br[...]
    hh = jnp.dot(rst, wh[...], preferred_element_type=f32) + bh[...]
    lzm = lz[...]
    z = jax.nn.sigmoid(jnp.dot(hz, lzm[:_D], preferred_element_type=f32)
                       + jnp.dot(h, lzm[_D:], preferred_element_type=f32)
                       + lzb[...])
    lrm = lr[...]
    r = jax.nn.sigmoid(jnp.dot(hr, lrm[:_D], preferred_element_type=f32)
                       + jnp.dot(h, lrm[_D:], preferred_element_type=f32)
                       + lrb[...])
    lhm = lh[...]
    ht = jnp.tanh(jnp.dot(hh, lhm[:_D], preferred_element_type=f32)
                  + jnp.dot(h * r, lhm[_D:], preferred_element_type=f32)
                  + lhb[...])
    oref[...] = z * h + (1.0 - z) * ht


def _full(shape):
    return pl.BlockSpec(shape, lambda i: tuple(0 for _ in shape))


_fin_call = pl.pallas_call(
    _fin_body,
    grid=(_NP // 128,),
    in_specs=[
        pl.BlockSpec((2, 128, _D), lambda i: (0, i, 0)),
        pl.BlockSpec((128, 1), lambda i: (i, 0)),
        pl.BlockSpec((128, _D), lambda i: (i, 0)),
        _full((_D, _D)), _full((1, _D)),
        _full((_D, _D)), _full((1, _D)),
        _full((_D, _D)), _full((1, _D)),
        _full((2 * _D, _D)), _full((1, _D)),
        _full((2 * _D, _D)), _full((1, _D)),
        _full((2 * _D, _D)), _full((1, _D)),
    ],
    out_specs=pl.BlockSpec((128, _D), lambda i: (i, 0)),
    out_shape=jax.ShapeDtypeStruct((_NP, _D), jnp.float32),
)


@functools.cache
def _sc_calls():
    mesh = plsc.VectorSubcoreMesh(
        core_axis_name="c", subcore_axis_name="s", num_cores=_NCORE,
        num_subcores=_NSUB)
    cparams = pltpu.CompilerParams(needs_layout_passes=False)
    deg_call = pl.kernel(
        _deg_body,
        out_type=jax.ShapeDtypeStruct((_NCORE, 2, _NP), jnp.float32),
        mesh=mesh,
        compiler_params=cparams,
        scratch_types=[
            pltpu.VMEM((_CPT, 2, _K), jnp.int32),
            pltpu.VMEM((_CPT * _K,), jnp.float32),
            pltpu.VMEM((_NP,), jnp.float32),
            pltpu.VMEM((_NP,), jnp.float32),
            pltpu.VMEM_SHARED((_NSUB, 2, _NP), jnp.float32),
            pltpu.VMEM((2, _PER_SUB), jnp.float32),
            pltpu.VMEM((2, _PER_SUB), jnp.float32),
        ],
    )
    agg_call = pl.kernel(
        _agg_body,
        out_type=jax.ShapeDtypeStruct((_NCORE, _NP, _D), jnp.float32),
        mesh=mesh,
        compiler_params=cparams,
        scratch_types=[
            pltpu.VMEM((_DR, 2, _K), jnp.int32),
            pltpu.VMEM((_DR, _K), jnp.float32),
            pltpu.VMEM((_NB, _K, _D), jnp.float32),
            pltpu.VMEM_SHARED((_NP, _D), jnp.float32),
        ] + [pltpu.SemaphoreType.DMA] * (2 * _NB + 2 * _DR),
    )
    return deg_call, agg_call


def kernel(X, edge_index, edge_weight, H, Wz, bz, Wr, br, Wh, bh,
           Lz_w, Lz_b, Lr_w, Lr_b, Lh_w, Lh_b):
    _deg_call, _agg_call = _sc_calls()
    pe = _EP - _E
    srcp = jnp.concatenate([edge_index[0], jnp.zeros((pe,), jnp.int32)])
    dstp = jnp.concatenate([edge_index[1], jnp.zeros((pe,), jnp.int32)])
    ewp = jnp.concatenate([edge_weight, jnp.zeros((pe,), jnp.float32)])
    edata = jnp.stack(
        [srcp.reshape(-1, _K), dstp.reshape(-1, _K)], axis=1)
    Xp = jnp.pad(X, ((0, _NP - _N), (0, 0)))
    Hp = jnp.pad(H, ((0, _NP - _N), (0, 0)))

    degs = _deg_call(edata, ewp)
    xs, ndst = _prep_call(degs.reshape(4, _NP, 1), Xp)
    agg = _agg_call(xs, edata, ewp)

    rb = lambda b: b.reshape(1, _D)
    out = _fin_call(agg, ndst, Hp, Wz, rb(bz), Wr, rb(br), Wh, rb(bh),
                    Lz_w, rb(Lz_b), Lr_w, rb(Lr_b), Lh_w, rb(Lh_b))
    return out[:_N]


# speedup vs baseline: 4.0826x; 1.1608x over previous
import functools

import jax
import jax.numpy as jnp
from jax import lax
from jax.experimental import pallas as pl
from jax.experimental.pallas import tpu as pltpu
from jax.experimental.pallas import tpu_sc as plsc

_N = 10000
_E = 320000
_D = 128
_NP = 10240
_EP = 327680
_NCORE = 2
_NSUB = 16
_TPE = _EP // (_NCORE * _NSUB)
_KA = 64
_CA = _TPE // _KA
_PER_SUB = _NP // _NSUB

_NB = 4
_DR = 8
_U = 8


def _deg_body(edata, ew, out, edv, ewv, deg_o, deg_i, slab, acc, tbuf):
    c = lax.axis_index("c")
    s = lax.axis_index("s")
    wid = s * _NCORE + c
    pltpu.sync_copy(edata.at[pl.ds(wid * _CA, _CA)], edv)
    pltpu.sync_copy(ew.at[pl.ds(wid * _TPE, _TPE)], ewv)

    zero = jnp.zeros((16,), jnp.float32)

    def zb(i, _):
        deg_o[pl.ds(i * 16, 16)] = zero
        deg_i[pl.ds(i * 16, 16)] = zero
        return 0
    lax.fori_loop(0, _NP // 16, zb, 0)

    def eb(m, _):
        for i in range(_KA // 16):
            sl = pl.ds(i * 16, 16)
            si = edv[m, 0, sl]
            di = edv[m, 1, sl]
            w = ewv[pl.ds(m * _KA + i * 16, 16)]
            plsc.addupdate_scatter(deg_o, [si], w)
            plsc.addupdate_scatter(deg_i, [di], w)
        return 0
    lax.fori_loop(0, _CA, eb, 0)

    pltpu.sync_copy(deg_o, slab.at[s, 0])
    pltpu.sync_copy(deg_i, slab.at[s, 1])
    plsc.subcore_barrier()

    nb = _PER_SUB // 16

    def za(i, _):
        acc[0, pl.ds(i * 16, 16)] = zero
        acc[1, pl.ds(i * 16, 16)] = zero
        return 0
    lax.fori_loop(0, nb, za, 0)

    def rb(t, _):
        pltpu.sync_copy(slab.at[t, :, pl.ds(s * _PER_SUB, _PER_SUB)], tbuf)

        def ab(b, _):
            sl = pl.ds(b * 16, 16)
            acc[0, sl] = acc[0, sl] + tbuf[0, sl]
            acc[1, sl] = acc[1, sl] + tbuf[1, sl]
            return 0
        lax.fori_loop(0, nb, ab, 0)
        return 0
    lax.fori_loop(0, _NSUB, rb, 0)

    pltpu.sync_copy(acc, out.at[c, :, pl.ds(s * _PER_SUB, _PER_SUB)])


def _agg_body(xs, edata, ew, out, desc, ewr, rows, agg_sh, *sems):
    c = lax.axis_index("c")
    s = lax.axis_index("s")
    wid = s * _NCORE + c
    sem_g = sems[0:_NB]
    sem_s = sems[_NB:2 * _NB]
    sem_d = sems[2 * _NB:2 * _NB + _DR]
    sem_w = sems[2 * _NB + _DR:2 * _NB + 2 * _DR]
    cbase = wid * _CA
    niter = _CA // _U

    zero = jnp.zeros((16,), jnp.float32)

    def zb(j, _):
        for v in range(_D // 16):
            rows[0, j, pl.ds(v * 16, 16)] = zero
        return 0
    lax.fori_loop(0, _KA, zb, 0)
    for jj in range(_PER_SUB // _KA):
        pltpu.sync_copy(rows.at[0],
                        agg_sh.at[pl.ds(s * _PER_SUB + jj * _KA, _KA)])
    plsc.subcore_barrier()

    def desc_cp(m, d):
        return pltpu.make_async_copy(edata.at[cbase + m], desc.at[d],
                                     sem_d[d])

    def ew_cp(m, d):
        return pltpu.make_async_copy(ew.at[pl.ds((cbase + m) * _KA, _KA)],
                                     ewr.at[d], sem_w[d])

    def gather(m, b, d):
        return pltpu.make_async_copy(xs.at[desc.at[d, 0]], rows.at[b],
                                     sem_g[b])

    def scatter(m, b, d):
        return pltpu.make_async_copy(rows.at[b], agg_sh.at[desc.at[d, 1]],
                                     sem_s[b])

    for j in range(_DR - 1):
        desc_cp(j, j).start()
        ew_cp(j, j).start()
    for j in range(_NB - 1):
        desc_cp(j, j).wait()
        gather(j, j, j).start()

    def pb(p, _):
        for q in range(_U):
            m = p * _U + q
            b = q % _NB
            d = q
            gather(m, b, d).wait()

            ew_cp(m, d).wait()

            def jb(j, _):
                w = plsc.load_gather(ewr.at[d],
                                     [jnp.full((16,), j, jnp.int32)])
                for v in range(_D // 16):
                    sl = pl.ds(v * 16, 16)
                    rows[b, j, sl] = rows[b, j, sl] * w
                return 0
            lax.fori_loop(0, _KA, jb, 0)

            def adv():
                scatter(m - 1, (q - 1) % _NB, (q - 1) % _DR).wait()

            if q > 0:
                adv()
            else:
                pl.when(p >= 1)(adv)

            def nxt():
                dn = (q + _NB - 1) % _DR
                desc_cp(m + _NB - 1, dn).wait()
                gather(m + _NB - 1, (q + _NB - 1) % _NB, dn).start()

            if q < _U - (_NB - 1):
                nxt()
            else:
                pl.when(p < niter - 1)(nxt)

            scatter(m, b, d).start(add=True)

            def refill():
                desc_cp(m + _DR - 1, (q + _DR - 1) % _DR).start()
                ew_cp(m + _DR - 1, (q + _DR - 1) % _DR).start()

            if q == 0:
                refill()
            else:
                pl.when(p < niter - 1)(refill)
        return 0
    lax.fori_loop(0, niter, pb, 0)
    scatter(_CA - 1, (_CA - 1) % _NB, (_CA - 1) % _DR).wait()

    plsc.subcore_barrier()
    sl = pl.ds(s * _PER_SUB, _PER_SUB)
    pltpu.sync_copy(agg_sh.at[sl], out.at[c, sl])


_RB = 1024


def _prep_body(dref, xref, xsref, ndref):
    d = dref[...]
    dego = d[0] + d[2]
    degi = d[1] + d[3]
    nsrc = jnp.where(dego > 0, lax.rsqrt(jnp.where(dego > 0, dego, 1.0)), 0.0)
    ndst = jnp.where(degi > 0, lax.rsqrt(jnp.where(degi > 0, degi, 1.0)), 0.0)
    xsref[...] = xref[...] * nsrc
    ndref[...] = ndst


_prep_call = pl.pallas_call(
    _prep_body,
    grid=(_NP // _RB,),
    in_specs=[
        pl.BlockSpec((4, _RB, 1), lambda i: (0, i, 0)),
        pl.BlockSpec((_RB, _D), lambda i: (i, 0)),
    ],
    out_specs=[
        pl.BlockSpec((_RB, _D), lambda i: (i, 0)),
        pl.BlockSpec((_RB, 1), lambda i: (i, 0)),
    ],
    out_shape=[
        jax.ShapeDtypeStruct((_NP, _D), jnp.float32),
        jax.ShapeDtypeStruct((_NP, 1), jnp.float32),
    ],
)


def _fin_body(aref, ndref, href, wz, bz, wr, br, wh, bh,
              lz, lzb, lr, lrb, lh, lhb, oref):
    f32 = jnp.float32
    agg = aref[0] + aref[1]
    rst = agg * ndref[...]
    h = href[...]
    hz = jnp.dot(rst, wz[...], preferred_element_type=f32) + bz[...]
    hr = jnp.dot(rst, wr[...], preferred_element_type=f32) + br[...]
    hh = jnp.dot(rst, wh[...], preferred_element_type=f32) + bh[...]
    lzm = lz[...]
    z = jax.nn.sigmoid(jnp.dot(hz, lzm[:_D], preferred_element_type=f32)
                       + jnp.dot(h, lzm[_D:], preferred_element_type=f32)
                       + lzb[...])
    lrm = lr[...]
    r = jax.nn.sigmoid(jnp.dot(hr, lrm[:_D], preferred_element_type=f32)
                       + jnp.dot(h, lrm[_D:], preferred_element_type=f32)
                       + lrb[...])
    lhm = lh[...]
    ht = jnp.tanh(jnp.dot(hh, lhm[:_D], preferred_element_type=f32)
                  + jnp.dot(h * r, lhm[_D:], preferred_element_type=f32)
                  + lhb[...])
    oref[...] = z * h + (1.0 - z) * ht


def _full(shape):
    return pl.BlockSpec(shape, lambda i: tuple(0 for _ in shape))


_fin_call = pl.pallas_call(
    _fin_body,
    grid=(_NP // _RB,),
    in_specs=[
        pl.BlockSpec((2, _RB, _D), lambda i: (0, i, 0)),
        pl.BlockSpec((_RB, 1), lambda i: (i, 0)),
        pl.BlockSpec((_RB, _D), lambda i: (i, 0)),
        _full((_D, _D)), _full((1, _D)),
        _full((_D, _D)), _full((1, _D)),
        _full((_D, _D)), _full((1, _D)),
        _full((2 * _D, _D)), _full((1, _D)),
        _full((2 * _D, _D)), _full((1, _D)),
        _full((2 * _D, _D)), _full((1, _D)),
    ],
    out_specs=pl.BlockSpec((_RB, _D), lambda i: (i, 0)),
    out_shape=jax.ShapeDtypeStruct((_NP, _D), jnp.float32),
)


@functools.cache
def _sc_calls():
    mesh = plsc.VectorSubcoreMesh(
        core_axis_name="c", subcore_axis_name="s", num_cores=_NCORE,
        num_subcores=_NSUB)
    cparams = pltpu.CompilerParams(needs_layout_passes=False)
    deg_call = pl.kernel(
        _deg_body,
        out_type=jax.ShapeDtypeStruct((_NCORE, 2, _NP), jnp.float32),
        mesh=mesh,
        compiler_params=cparams,
        scratch_types=[
            pltpu.VMEM((_CA, 2, _KA), jnp.int32),
            pltpu.VMEM((_TPE,), jnp.float32),
            pltpu.VMEM((_NP,), jnp.float32),
            pltpu.VMEM((_NP,), jnp.float32),
            pltpu.VMEM_SHARED((_NSUB, 2, _NP), jnp.float32),
            pltpu.VMEM((2, _PER_SUB), jnp.float32),
            pltpu.VMEM((2, _PER_SUB), jnp.float32),
        ],
    )
    agg_call = pl.kernel(
        _agg_body,
        out_type=jax.ShapeDtypeStruct((_NCORE, _NP, _D), jnp.float32),
        mesh=mesh,
        compiler_params=cparams,
        scratch_types=[
            pltpu.VMEM((_DR, 2, _KA), jnp.int32),
            pltpu.VMEM((_DR, _KA), jnp.float32),
            pltpu.VMEM((_NB, _KA, _D), jnp.float32),
            pltpu.VMEM_SHARED((_NP, _D), jnp.float32),
        ] + [pltpu.SemaphoreType.DMA] * (2 * _NB + 2 * _DR),
    )
    return deg_call, agg_call


def kernel(X, edge_index, edge_weight, H, Wz, bz, Wr, br, Wh, bh,
           Lz_w, Lz_b, Lr_w, Lr_b, Lh_w, Lh_b):
    _deg_call, _agg_call = _sc_calls()
    pe = _EP - _E
    srcp = jnp.concatenate([edge_index[0], jnp.zeros((pe,), jnp.int32)])
    dstp = jnp.concatenate([edge_index[1], jnp.zeros((pe,), jnp.int32)])
    ewp = jnp.concatenate([edge_weight, jnp.zeros((pe,), jnp.float32)])
    edata = jnp.stack(
        [srcp.reshape(-1, _KA), dstp.reshape(-1, _KA)], axis=1)
    Xp = jnp.pad(X, ((0, _NP - _N), (0, 0)))
    Hp = jnp.pad(H, ((0, _NP - _N), (0, 0)))

    degs = _deg_call(edata, ewp)
    xs, ndst = _prep_call(degs.reshape(4, _NP, 1), Xp)
    agg = _agg_call(xs, edata, ewp)

    rb = lambda b: b.reshape(1, _D)
    out = _fin_call(agg, ndst, Hp, Wz, rb(bz), Wr, rb(br), Wh, rb(bh),
                    Lz_w, rb(Lz_b), Lr_w, rb(Lr_b), Lh_w, rb(Lh_b))
    return out[:_N]


# speedup vs baseline: 4.4436x; 1.0884x over previous
import functools

import jax
import jax.numpy as jnp
from jax import lax
from jax.experimental import pallas as pl
from jax.experimental.pallas import tpu as pltpu
from jax.experimental.pallas import tpu_sc as plsc

_N = 10000
_E = 320000
_D = 128
_NP = 10240
_G = _NP // 128
_EP = 327680
_NCORE = 2
_NSUB = 16
_TPE = _EP // (_NCORE * _NSUB)
_KA = 64
_NCH = _EP // _KA
_PER_SUB = _NP // _NSUB

_CA0 = 248
_CA1 = 72

_NB = 4
_DR = 8
_U = 8


def _deg_body(edata, ew, out, edv, ewv, deg_o, deg_i, slab, acc, tbuf):
    c = lax.axis_index("c")
    s = lax.axis_index("s")
    wid = s * _NCORE + c
    ca = _TPE // _KA
    pltpu.sync_copy(edata.at[pl.ds(wid * ca, ca)], edv)
    pltpu.sync_copy(ew.at[pl.ds(wid * _TPE, _TPE)], ewv)

    zero = jnp.zeros((16,), jnp.float32)

    def zb(i, _):
        deg_o[pl.ds(i * 16, 16)] = zero
        deg_i[pl.ds(i * 16, 16)] = zero
        return 0
    lax.fori_loop(0, _NP // 16, zb, 0)

    def eb(m, _):
        for i in range(_KA // 16):
            sl = pl.ds(i * 16, 16)
            si = edv[m, 0, sl]
            di = edv[m, 1, sl]
            w = ewv[pl.ds(m * _KA + i * 16, 16)]
            plsc.addupdate_scatter(deg_o, [si], w)
            plsc.addupdate_scatter(deg_i, [di], w)
        return 0
    lax.fori_loop(0, ca, eb, 0)

    pltpu.sync_copy(deg_o, slab.at[s, 0])
    pltpu.sync_copy(deg_i, slab.at[s, 1])
    plsc.subcore_barrier()

    def za(i, _):
        acc[0, pl.ds(i * 16, 16)] = zero
        acc[1, pl.ds(i * 16, 16)] = zero
        return 0
    lax.fori_loop(0, _PER_SUB // 16, za, 0)

    def rb(t, _):
        pltpu.sync_copy(slab.at[t, :, pl.ds(s * _PER_SUB, _PER_SUB)], tbuf)

        def ab(b, _):
            sl = pl.ds(b * 16, 16)
            acc[0, sl] = acc[0, sl] + tbuf[0, sl]
            acc[1, sl] = acc[1, sl] + tbuf[1, sl]
            return 0
        lax.fori_loop(0, _PER_SUB // 16, ab, 0)
        return 0
    lax.fori_loop(0, _NSUB, rb, 0)

    pltpu.sync_copy(acc, out.at[c, :, pl.ds(s * _PER_SUB, _PER_SUB)])


def _agg_body(xs, edata, ew, out, desc, ewr, rows, agg_sh, *sems):
    c = lax.axis_index("c")
    s = lax.axis_index("s")
    sem_g = sems[0:_NB]
    sem_s = sems[_NB:2 * _NB]
    sem_d = sems[2 * _NB:2 * _NB + _DR]
    sem_w = sems[2 * _NB + _DR:2 * _NB + 2 * _DR]

    zero = jnp.zeros((16,), jnp.float32)

    def zb(j, _):
        for v in range(_D // 16):
            rows[0, j, pl.ds(v * 16, 16)] = zero
        return 0
    lax.fori_loop(0, _KA, zb, 0)
    for jj in range(_PER_SUB // _KA):
        pltpu.sync_copy(rows.at[0],
                        agg_sh.at[pl.ds(s * _PER_SUB + jj * _KA, _KA)])
    plsc.subcore_barrier()

    def pipeline(cbase, ca):
        niter = ca // _U

        def desc_cp(m, d):
            return pltpu.make_async_copy(edata.at[cbase + m], desc.at[d],
                                         sem_d[d])

        def ew_cp(m, d):
            return pltpu.make_async_copy(ew.at[pl.ds((cbase + m) * _KA, _KA)],
                                         ewr.at[d], sem_w[d])

        def gather(m, b, d):
            return pltpu.make_async_copy(xs.at[desc.at[d, 0]], rows.at[b],
                                         sem_g[b])

        def scatter(m, b, d):
            return pltpu.make_async_copy(rows.at[b],
                                         agg_sh.at[desc.at[d, 1]], sem_s[b])

        for j in range(_DR - 1):
            desc_cp(j, j).start()
            ew_cp(j, j).start()
        for j in range(_NB - 1):
            desc_cp(j, j).wait()
            gather(j, j, j).start()

        def pb(p, _):
            for q in range(_U):
                m = p * _U + q
                b = q % _NB
                d = q
                gather(m, b, d).wait()

                ew_cp(m, d).wait()

                def jb(j, _):
                    w = plsc.load_gather(ewr.at[d],
                                         [jnp.full((16,), j, jnp.int32)])
                    for v in range(_D // 16):
                        sl = pl.ds(v * 16, 16)
                        rows[b, j, sl] = rows[b, j, sl] * w
                    return 0
                lax.fori_loop(0, _KA, jb, 0)

                def adv():
                    scatter(m - 1, (q - 1) % _NB, (q - 1) % _DR).wait()

                if q > 0:
                    adv()
                else:
                    pl.when(p >= 1)(adv)

                def nxt():
                    dn = (q + _NB - 1) % _DR
                    desc_cp(m + _NB - 1, dn).wait()
                    gather(m + _NB - 1, (q + _NB - 1) % _NB, dn).start()

                if q < _U - (_NB - 1):
                    nxt()
                else:
                    pl.when(p < niter - 1)(nxt)

                scatter(m, b, d).start(add=True)

                def refill():
                    desc_cp(m + _DR - 1, (q + _DR - 1) % _DR).start()
                    ew_cp(m + _DR - 1, (q + _DR - 1) % _DR).start()

                if q == 0:
                    refill()
                else:
                    pl.when(p < niter - 1)(refill)
            return 0
        lax.fori_loop(0, niter, pb, 0)
        scatter(ca - 1, (ca - 1) % _NB, (ca - 1) % _DR).wait()

    pl.when(c == 0)(lambda: pipeline(s * _CA0, _CA0))
    pl.when(c != 0)(lambda: pipeline(_NSUB * _CA0 + s * _CA1, _CA1))

    plsc.subcore_barrier()
    sl = pl.ds(s * _PER_SUB, _PER_SUB)
    pltpu.sync_copy(agg_sh.at[sl], out.at[c, sl])


_RB = 1024
_RG = _RB // 128


def _eye():
    r = lax.broadcasted_iota(jnp.int32, (_D, _D), 0)
    cc = lax.broadcasted_iota(jnp.int32, (_D, _D), 1)
    return jnp.where(r == cc, 1.0, 0.0).astype(jnp.float32)


def _row_scale(nmat, x):
    eye = _eye()
    outs = []
    for r in range(_RG):
        dmat = nmat[r:r + 1, :] * eye
        outs.append(jnp.dot(dmat, x[r * 128:(r + 1) * 128],
                            preferred_element_type=jnp.float32))
    return jnp.concatenate(outs, axis=0)


def _prep_body(dref, xref, xsref, ndref):
    d = dref[...]
    dego = d[0] + d[2]
    degi = d[1] + d[3]
    nsrc = jnp.where(dego > 0, lax.rsqrt(jnp.where(dego > 0, dego, 1.0)), 0.0)
    ndst = jnp.where(degi > 0, lax.rsqrt(jnp.where(degi > 0, degi, 1.0)), 0.0)
    ndref[...] = ndst
    xsref[...] = _row_scale(nsrc, xref[...])


_prep_call = pl.pallas_call(
    _prep_body,
    grid=(_NP // _RB,),
    in_specs=[
        pl.BlockSpec((4, _RG, _D), lambda i: (0, i, 0)),
        pl.BlockSpec((_RB, _D), lambda i: (i, 0)),
    ],
    out_specs=[
        pl.BlockSpec((_RB, _D), lambda i: (i, 0)),
        pl.BlockSpec((_RG, _D), lambda i: (i, 0)),
    ],
    out_shape=[
        jax.ShapeDtypeStruct((_NP, _D), jnp.float32),
        jax.ShapeDtypeStruct((_G, _D), jnp.float32),
    ],
)


def _fin_body(aref, ndref, href, wz, bz, wr, br, wh, bh,
              lz, lzb, lr, lrb, lh, lhb, oref):
    f32 = jnp.float32
    agg = aref[0] + aref[1]
    rst = _row_scale(ndref[...], agg)
    h = href[...]
    hz = jnp.dot(rst, wz[...], preferred_element_type=f32) + bz[...]
    hr = jnp.dot(rst, wr[...], preferred_element_type=f32) + br[...]
    hh = jnp.dot(rst, wh[...], preferred_element_type=f32) + bh[...]
    lzm = lz[...]
    z = jax.nn.sigmoid(jnp.dot(hz, lzm[:_D], preferred_element_type=f32)
                       + jnp.dot(h, lzm[_D:], preferred_element_type=f32)
                       + lzb[...])
    lrm = lr[...]
    r = jax.nn.sigmoid(jnp.dot(hr, lrm[:_D], preferred_element_type=f32)
                       + jnp.dot(h, lrm[_D:], preferred_element_type=f32)
                       + lrb[...])
    lhm = lh[...]
    ht = jnp.tanh(jnp.dot(hh, lhm[:_D], preferred_element_type=f32)
                  + jnp.dot(h * r, lhm[_D:], preferred_element_type=f32)
                  + lhb[...])
    oref[...] = z * h + (1.0 - z) * ht


def _full(shape):
    return pl.BlockSpec(shape, lambda i: tuple(0 for _ in shape))


_fin_call = pl.pallas_call(
    _fin_body,
    grid=(_NP // _RB,),
    in_specs=[
        pl.BlockSpec((2, _RB, _D), lambda i: (0, i, 0)),
        pl.BlockSpec((_RG, _D), lambda i: (i, 0)),
        pl.BlockSpec((_RB, _D), lambda i: (i, 0)),
        _full((_D, _D)), _full((1, _D)),
        _full((_D, _D)), _full((1, _D)),
        _full((_D, _D)), _full((1, _D)),
        _full((2 * _D, _D)), _full((1, _D)),
        _full((2 * _D, _D)), _full((1, _D)),
        _full((2 * _D, _D)), _full((1, _D)),
    ],
    out_specs=pl.BlockSpec((_RB, _D), lambda i: (i, 0)),
    out_shape=jax.ShapeDtypeStruct((_NP, _D), jnp.float32),
)


@functools.cache
def _sc_calls():
    mesh = plsc.VectorSubcoreMesh(
        core_axis_name="c", subcore_axis_name="s", num_cores=_NCORE,
        num_subcores=_NSUB)
    cparams = pltpu.CompilerParams(needs_layout_passes=False)
    ca = _TPE // _KA
    deg_call = pl.kernel(
        _deg_body,
        out_type=jax.ShapeDtypeStruct((_NCORE, 2, _NP), jnp.float32),
        mesh=mesh,
        compiler_params=cparams,
        scratch_types=[
            pltpu.VMEM((ca, 2, _KA), jnp.int32),
            pltpu.VMEM((_TPE,), jnp.float32),
            pltpu.VMEM((_NP,), jnp.float32),
            pltpu.VMEM((_NP,), jnp.float32),
            pltpu.VMEM_SHARED((_NSUB, 2, _NP), jnp.float32),
            pltpu.VMEM((2, _PER_SUB), jnp.float32),
            pltpu.VMEM((2, _PER_SUB), jnp.float32),
        ],
    )
    agg_call = pl.kernel(
        _agg_body,
        out_type=jax.ShapeDtypeStruct((_NCORE, _NP, _D), jnp.float32),
        mesh=mesh,
        compiler_params=cparams,
        scratch_types=[
            pltpu.VMEM((_DR, 2, _KA), jnp.int32),
            pltpu.VMEM((_DR, _KA), jnp.float32),
            pltpu.VMEM((_NB, _KA, _D), jnp.float32),
            pltpu.VMEM_SHARED((_NP, _D), jnp.float32),
        ] + [pltpu.SemaphoreType.DMA] * (2 * _NB + 2 * _DR),
    )
    return deg_call, agg_call


def kernel(X, edge_index, edge_weight, H, Wz, bz, Wr, br, Wh, bh,
           Lz_w, Lz_b, Lr_w, Lr_b, Lh_w, Lh_b):
    _deg_call, _agg_call = _sc_calls()
    pe = _EP - _E
    srcp = jnp.concatenate([edge_index[0], jnp.zeros((pe,), jnp.int32)])
    dstp = jnp.concatenate([edge_index[1], jnp.zeros((pe,), jnp.int32)])
    ewp = jnp.concatenate([edge_weight, jnp.zeros((pe,), jnp.float32)])
    edata = jnp.stack(
        [srcp.reshape(-1, _KA), dstp.reshape(-1, _KA)], axis=1)
    Xp = jnp.pad(X, ((0, _NP - _N), (0, 0)))
    Hp = jnp.pad(H, ((0, _NP - _N), (0, 0)))

    degs = _deg_call(edata, ewp)
    xs, ndst = _prep_call(degs.reshape(4, _G, _D), Xp)
    agg = _agg_call(xs, edata, ewp)

    rb = lambda b: b.reshape(1, _D)
    out = _fin_call(agg, ndst, Hp, Wz, rb(bz), Wr, rb(br), Wh, rb(bh),
                    Lz_w, rb(Lz_b), Lr_w, rb(Lr_b), Lh_w, rb(Lh_b))
    return out[:_N]


# speedup vs baseline: 5.0454x; 1.1354x over previous
import functools

import jax
import jax.numpy as jnp
from jax import lax
from jax.experimental import pallas as pl
from jax.experimental.pallas import tpu as pltpu
from jax.experimental.pallas import tpu_sc as plsc

_N = 10000
_E = 320000
_D = 128
_NP = 10240
_G = _NP // 128
_EP = 327680
_NCORE = 2
_NSUB = 16
_TPE = _EP // (_NCORE * _NSUB)
_KA = 64
_NCH = _EP // _KA
_PER_SUB = _NP // _NSUB

_CA0 = 288
_CA1 = 32

_CD0 = 208
_CD1 = 112

_NB = 4
_DR = 8
_U = 8


def _deg_body(srcp, dstp, ew, out, sv, dv, ewv, deg_o, deg_i, slab, acc,
              tbuf):
    c = lax.axis_index("c")
    s = lax.axis_index("s")

    zero = jnp.zeros((16,), jnp.float32)

    def zb(i, _):
        deg_o[pl.ds(i * 16, 16)] = zero
        deg_i[pl.ds(i * 16, 16)] = zero
        return 0
    lax.fori_loop(0, _NP // 16, zb, 0)

    wid = s * _NCORE + c
    ebase = wid * _TPE
    pltpu.sync_copy(srcp.at[pl.ds(ebase, _TPE)], sv.at[pl.ds(0, _TPE)])
    pltpu.sync_copy(dstp.at[pl.ds(ebase, _TPE)], dv.at[pl.ds(0, _TPE)])
    pltpu.sync_copy(ew.at[pl.ds(ebase, _TPE)], ewv.at[pl.ds(0, _TPE)])

    def eb(i, _):
        sl = pl.ds(i * 16, 16)
        plsc.addupdate_scatter(deg_o, [sv[sl]], ewv[sl])
        plsc.addupdate_scatter(deg_i, [dv[sl]], ewv[sl])
        return 0
    lax.fori_loop(0, _TPE // 16, eb, 0)

    pltpu.sync_copy(deg_o, slab.at[s, 0])
    pltpu.sync_copy(deg_i, slab.at[s, 1])
    plsc.subcore_barrier()

    def za(i, _):
        acc[0, pl.ds(i * 16, 16)] = zero
        acc[1, pl.ds(i * 16, 16)] = zero
        return 0
    lax.fori_loop(0, _PER_SUB // 16, za, 0)

    def rb(t, _):
        pltpu.sync_copy(slab.at[t, :, pl.ds(s * _PER_SUB, _PER_SUB)], tbuf)

        def ab(b, _):
            sl = pl.ds(b * 16, 16)
            acc[0, sl] = acc[0, sl] + tbuf[0, sl]
            acc[1, sl] = acc[1, sl] + tbuf[1, sl]
            return 0
        lax.fori_loop(0, _PER_SUB // 16, ab, 0)
        return 0
    lax.fori_loop(0, _NSUB, rb, 0)

    pltpu.sync_copy(acc, out.at[c, :, pl.ds(s * _PER_SUB, _PER_SUB)])


def _agg_body(xs, edata, ew, out, desc, ewr, rows, agg_sh, *sems):
    c = lax.axis_index("c")
    s = lax.axis_index("s")
    sem_g = sems[0:_NB]
    sem_s = sems[_NB:2 * _NB]
    sem_d = sems[2 * _NB:2 * _NB + _DR]
    sem_w = sems[2 * _NB + _DR:2 * _NB + 2 * _DR]

    zero = jnp.zeros((16,), jnp.float32)

    def zb(j, _):
        for v in range(_D // 16):
            rows[0, j, pl.ds(v * 16, 16)] = zero
        return 0
    lax.fori_loop(0, _KA, zb, 0)
    for jj in range(_PER_SUB // _KA):
        pltpu.sync_copy(rows.at[0],
                        agg_sh.at[pl.ds(s * _PER_SUB + jj * _KA, _KA)])
    plsc.subcore_barrier()

    def pipeline(cbase, ca):
        niter = ca // _U

        def desc_cp(m, d):
            return pltpu.make_async_copy(edata.at[cbase + m], desc.at[d],
                                         sem_d[d])

        def ew_cp(m, d):
            return pltpu.make_async_copy(ew.at[pl.ds((cbase + m) * _KA, _KA)],
                                         ewr.at[d], sem_w[d])

        def gather(m, b, d):
            return pltpu.make_async_copy(xs.at[desc.at[d, 0]], rows.at[b],
                                         sem_g[b])

        def scatter(m, b, d):
            return pltpu.make_async_copy(rows.at[b],
                                         agg_sh.at[desc.at[d, 1]], sem_s[b])

        for j in range(_DR - 1):
            desc_cp(j, j).start()
            ew_cp(j, j).start()
        for j in range(_NB - 1):
            desc_cp(j, j).wait()
            gather(j, j, j).start()

        def pb(p, _):
            for q in range(_U):
                m = p * _U + q
                b = q % _NB
                d = q
                gather(m, b, d).wait()

                ew_cp(m, d).wait()

                def jb(j, _):
                    w = plsc.load_gather(ewr.at[d],
                                         [jnp.full((16,), j, jnp.int32)])
                    for v in range(_D // 16):
                        sl = pl.ds(v * 16, 16)
                        rows[b, j, sl] = rows[b, j, sl] * w
                    return 0
                lax.fori_loop(0, _KA, jb, 0)

                def adv():
                    scatter(m - 1, (q - 1) % _NB, (q - 1) % _DR).wait()

                if q > 0:
                    adv()
                else:
                    pl.when(p >= 1)(adv)

                def nxt():
                    dn = (q + _NB - 1) % _DR
                    desc_cp(m + _NB - 1, dn).wait()
                    gather(m + _NB - 1, (q + _NB - 1) % _NB, dn).start()

                if q < _U - (_NB - 1):
                    nxt()
                else:
                    pl.when(p < niter - 1)(nxt)

                scatter(m, b, d).start(add=True)

                def refill():
                    desc_cp(m + _DR - 1, (q + _DR - 1) % _DR).start()
                    ew_cp(m + _DR - 1, (q + _DR - 1) % _DR).start()

                if q == 0:
                    refill()
                else:
                    pl.when(p < niter - 1)(refill)
            return 0
        lax.fori_loop(0, niter, pb, 0)
        scatter(ca - 1, (ca - 1) % _NB, (ca - 1) % _DR).wait()

    pl.when(c == 0)(lambda: pipeline(s * _CA0, _CA0))
    pl.when(c != 0)(lambda: pipeline(_NSUB * _CA0 + s * _CA1, _CA1))

    plsc.subcore_barrier()
    sl = pl.ds(s * _PER_SUB, _PER_SUB)
    pltpu.sync_copy(agg_sh.at[sl], out.at[c, sl])


_RB = 1024
_RG = _RB // 128


def _eye():
    r = lax.broadcasted_iota(jnp.int32, (_D, _D), 0)
    cc = lax.broadcasted_iota(jnp.int32, (_D, _D), 1)
    return jnp.where(r == cc, 1.0, 0.0).astype(jnp.float32)


def _row_scale(nmat, x):
    eye = _eye()
    outs = []
    for r in range(_RG):
        dmat = nmat[r:r + 1, :] * eye
        outs.append(jnp.dot(dmat, x[r * 128:(r + 1) * 128],
                            preferred_element_type=jnp.float32))
    return jnp.concatenate(outs, axis=0)


def _prep_body(dref, xref, xsref, ndref):
    d = dref[...]
    dego = d[0] + d[2]
    degi = d[1] + d[3]
    nsrc = jnp.where(dego > 0, lax.rsqrt(jnp.where(dego > 0, dego, 1.0)), 0.0)
    ndst = jnp.where(degi > 0, lax.rsqrt(jnp.where(degi > 0, degi, 1.0)), 0.0)
    ndref[...] = ndst
    xsref[...] = _row_scale(nsrc, xref[...])


_prep_call = pl.pallas_call(
    _prep_body,
    grid=(_NP // _RB,),
    in_specs=[
        pl.BlockSpec((4, _RG, _D), lambda i: (0, i, 0)),
        pl.BlockSpec((_RB, _D), lambda i: (i, 0)),
    ],
    out_specs=[
        pl.BlockSpec((_RB, _D), lambda i: (i, 0)),
        pl.BlockSpec((_RG, _D), lambda i: (i, 0)),
    ],
    out_shape=[
        jax.ShapeDtypeStruct((_NP, _D), jnp.float32),
        jax.ShapeDtypeStruct((_G, _D), jnp.float32),
    ],
)


def _fin_body(aref, ndref, href, wz, bz, wr, br, wh, bh,
              lz, lzb, lr, lrb, lh, lhb, oref):
    f32 = jnp.float32
    agg = aref[0] + aref[1]
    rst = _row_scale(ndref[...], agg)
    h = href[...]
    hz = jnp.dot(rst, wz[...], preferred_element_type=f32) + bz[...]
    hr = jnp.dot(rst, wr[...], preferred_element_type=f32) + br[...]
    hh = jnp.dot(rst, wh[...], preferred_element_type=f32) + bh[...]
    lzm = lz[...]
    z = jax.nn.sigmoid(jnp.dot(hz, lzm[:_D], preferred_element_type=f32)
                       + jnp.dot(h, lzm[_D:], preferred_element_type=f32)
                       + lzb[...])
    lrm = lr[...]
    r = jax.nn.sigmoid(jnp.dot(hr, lrm[:_D], preferred_element_type=f32)
                       + jnp.dot(h, lrm[_D:], preferred_element_type=f32)
                       + lrb[...])
    lhm = lh[...]
    ht = jnp.tanh(jnp.dot(hh, lhm[:_D], preferred_element_type=f32)
                  + jnp.dot(h * r, lhm[_D:], preferred_element_type=f32)
                  + lhb[...])
    oref[...] = z * h + (1.0 - z) * ht


def _full(shape):
    return pl.BlockSpec(shape, lambda i: tuple(0 for _ in shape))


_fin_call = pl.pallas_call(
    _fin_body,
    grid=(_NP // _RB,),
    in_specs=[
        pl.BlockSpec((2, _RB, _D), lambda i: (0, i, 0)),
        pl.BlockSpec((_RG, _D), lambda i: (i, 0)),
        pl.BlockSpec((_RB, _D), lambda i: (i, 0)),
        _full((_D, _D)), _full((1, _D)),
        _full((_D, _D)), _full((1, _D)),
        _full((_D, _D)), _full((1, _D)),
        _full((2 * _D, _D)), _full((1, _D)),
        _full((2 * _D, _D)), _full((1, _D)),
        _full((2 * _D, _D)), _full((1, _D)),
    ],
    out_specs=pl.BlockSpec((_RB, _D), lambda i: (i, 0)),
    out_shape=jax.ShapeDtypeStruct((_NP, _D), jnp.float32),
)


@functools.cache
def _sc_calls():
    mesh = plsc.VectorSubcoreMesh(
        core_axis_name="c", subcore_axis_name="s", num_cores=_NCORE,
        num_subcores=_NSUB)
    cparams = pltpu.CompilerParams(needs_layout_passes=False)
    deg_call = pl.kernel(
        _deg_body,
        out_type=jax.ShapeDtypeStruct((_NCORE, 2, _NP), jnp.float32),
        mesh=mesh,
        compiler_params=cparams,
        scratch_types=[
            pltpu.VMEM((_CD0 * _KA,), jnp.int32),
            pltpu.VMEM((_CD0 * _KA,), jnp.int32),
            pltpu.VMEM((_CD0 * _KA,), jnp.float32),
            pltpu.VMEM((_NP,), jnp.float32),
            pltpu.VMEM((_NP,), jnp.float32),
            pltpu.VMEM_SHARED((_NSUB, 2, _NP), jnp.float32),
            pltpu.VMEM((2, _PER_SUB), jnp.float32),
            pltpu.VMEM((2, _PER_SUB), jnp.float32),
        ],
    )
    agg_call = pl.kernel(
        _agg_body,
        out_type=jax.ShapeDtypeStruct((_NCORE, _NP, _D), jnp.float32),
        mesh=mesh,
        compiler_params=cparams,
        scratch_types=[
            pltpu.VMEM((_DR, 2, _KA), jnp.int32),
            pltpu.VMEM((_DR, _KA), jnp.float32),
            pltpu.VMEM((_NB, _KA, _D), jnp.float32),
            pltpu.VMEM_SHARED((_NP, _D), jnp.float32),
        ] + [pltpu.SemaphoreType.DMA] * (2 * _NB + 2 * _DR),
    )
    return deg_call, agg_call


def kernel(X, edge_index, edge_weight, H, Wz, bz, Wr, br, Wh, bh,
           Lz_w, Lz_b, Lr_w, Lr_b, Lh_w, Lh_b):
    _deg_call, _agg_call = _sc_calls()
    pe = _EP - _E
    srcp = jnp.concatenate([edge_index[0], jnp.zeros((pe,), jnp.int32)])
    dstp = jnp.concatenate([edge_index[1], jnp.zeros((pe,), jnp.int32)])
    ewp = jnp.concatenate([edge_weight, jnp.zeros((pe,), jnp.float32)])
    edata = jnp.stack(
        [srcp.reshape(-1, _KA), dstp.reshape(-1, _KA)], axis=1)
    Xp = jnp.pad(X, ((0, _NP - _N), (0, 0)))
    Hp = jnp.pad(H, ((0, _NP - _N), (0, 0)))

    degs = _deg_call(srcp, dstp, ewp)
    xs, ndst = _prep_call(degs.reshape(4, _G, _D), Xp)
    agg = _agg_call(xs, edata, ewp)

    rb = lambda b: b.reshape(1, _D)
    out = _fin_call(agg, ndst, Hp, Wz, rb(bz), Wr, rb(br), Wh, rb(bh),
                    Lz_w, rb(Lz_b), Lr_w, rb(Lr_b), Lh_w, rb(Lh_b))
    return out[:_N]
